# baseline (speedup 1.0000x reference)
import functools

import jax
import jax.numpy as jnp
from jax import lax
from jax.experimental import pallas as pl
from jax.experimental.pallas import tpu as pltpu
from jax.experimental.pallas import tpu_sc as plsc

N_NODES = 10000
N_EDGES = 160000
D = 256
H = 128
NE = 4
N_STEPS = 6

BN = 1000
NB = N_NODES // BN

N_SUBCORES = 16
CH = 128
NBUF = 2
NCH = -(-(N_EDGES // N_SUBCORES) // (CH * NBUF)) * NBUF
NOUT = NCH // NBUF
E_PAD = N_SUBCORES * NCH * CH
ZR = -(-(N_NODES + 1) // N_SUBCORES) // 8 * 8 + 8
N_ACC = N_SUBCORES * ZR


def _sc_body(key_hbm, dst_hbm, t0_hbm, t1_hbm, z_hbm, a_hbm,
             kbuf, dring, mbufs, acc, gsems, dsems):
    c = lax.axis_index("c")
    s = lax.axis_index("s")
    pltpu.sync_copy(key_hbm.at[s], kbuf)
    pltpu.sync_copy(z_hbm.at[pl.ds(s * ZR, ZR)], acc.at[pl.ds(s * ZR, ZR)])
    plsc.subcore_barrier()

    def run(t_hbm):
        def start(j, b):
            pltpu.async_copy(t_hbm.at[kbuf.at[j]], mbufs[b], gsems[b])
            pltpu.async_copy(dst_hbm.at[s].at[j], dring.at[b], dsems[b])

        def wait(b):
            pltpu.make_async_copy(t_hbm.at[kbuf.at[0]], mbufs[b],
                                  gsems[b]).wait()
            pltpu.make_async_copy(dst_hbm.at[s].at[0], dring.at[b],
                                  dsems[b]).wait()

        def scatter(b):
            pltpu.sync_copy(mbufs[b], acc.at[dring.at[b]], add=True)

        for b in range(NBUF):
            start(b, b)

        def round_(j0, carry):
            for b in range(NBUF):
                j = j0 * NBUF + b
                wait(b)
                scatter(b)
                start(j + NBUF, b)
            return carry

        lax.fori_loop(0, NOUT - 1, round_, 0)
        for b in range(NBUF):
            wait(b)
            scatter(b)

    @pl.when(c == 0)
    def _():
        run(t0_hbm)

    @pl.when(c == 1)
    def _():
        run(t1_hbm)

    plsc.subcore_barrier()

    @pl.when(c == 0)
    def _():
        pltpu.sync_copy(acc.at[pl.ds(s * ZR, ZR)],
                        a_hbm.at[0].at[pl.ds(s * ZR, ZR)])

    @pl.when(c == 1)
    def _():
        pltpu.sync_copy(acc.at[pl.ds(s * ZR, ZR)],
                        a_hbm.at[1].at[pl.ds(s * ZR, ZR)])


@functools.lru_cache(maxsize=1)
def _sc_scatter_call():
    mesh = plsc.VectorSubcoreMesh(core_axis_name="c", subcore_axis_name="s")
    return pl.kernel(
        _sc_body,
        out_type=jax.ShapeDtypeStruct((2, N_ACC, H), jnp.float32),
        mesh=mesh,
        scratch_types=[
            pltpu.VMEM((NCH, CH), jnp.int32),
            pltpu.VMEM((NBUF, CH), jnp.int32),
            [pltpu.VMEM((CH, H), jnp.float32)] * NBUF,
            pltpu.VMEM_SHARED((N_ACC, H), jnp.float32),
            [pltpu.SemaphoreType.DMA] * NBUF,
            [pltpu.SemaphoreType.DMA] * NBUF,
        ],
    )


def _transform_write(hn, wt_ref, bt_ref, t0_ref, t1_ref):
    for k in range(NE):
        v = jnp.dot(hn, wt_ref[k], preferred_element_type=jnp.float32)
        v = v + bt_ref[k]
        t0_ref[k] = v[:, :H]
        t1_ref[k] = v[:, H:]


def _gru_compute(h, a0, a1, wih_ref, whh_ref, bih_ref, bhh_ref):
    a = jnp.concatenate([a0, a1], axis=1)
    gi = jnp.dot(a, wih_ref[...], preferred_element_type=jnp.float32) + bih_ref[...]
    gh = jnp.dot(h, whh_ref[...], preferred_element_type=jnp.float32) + bhh_ref[...]
    r = jax.nn.sigmoid(gi[:, :D] + gh[:, :D])
    z = jax.nn.sigmoid(gi[:, D:2 * D] + gh[:, D:2 * D])
    n = jnp.tanh(gi[:, 2 * D:] + r * gh[:, 2 * D:])
    return (1.0 - z) * n + z * h


def _t_body(x_ref, wt_ref, bt_ref, t0_ref, t1_ref):
    _transform_write(x_ref[...], wt_ref, bt_ref, t0_ref, t1_ref)


def _a_body(h_ref, a_ref, wih_ref, whh_ref, bih_ref, bhh_ref, wt_ref, bt_ref,
            h_out, t0_ref, t1_ref):
    hn = _gru_compute(h_ref[...], a_ref[0], a_ref[1],
                      wih_ref, whh_ref, bih_ref, bhh_ref)
    h_out[...] = hn
    _transform_write(hn, wt_ref, bt_ref, t0_ref, t1_ref)


def _g_body(h_ref, a_ref, wih_ref, whh_ref, bih_ref, bhh_ref, h_out):
    h_out[...] = _gru_compute(h_ref[...], a_ref[0], a_ref[1],
                              wih_ref, whh_ref, bih_ref, bhh_ref)


def _p_body(h_ref, gw_ref, gb_ref, fwt_ref, fb_ref, cw_ref, cb_ref, out_ref):
    h = h_ref[...]
    gate = jnp.dot(h, gw_ref[...], preferred_element_type=jnp.float32) + gb_ref[...]
    m = jnp.max(gate)
    e = jnp.exp(gate - m)
    alpha = e / jnp.sum(e)
    feat = jnp.dot(h, fwt_ref[...],
                   preferred_element_type=jnp.float32) + fb_ref[...]
    hg = jnp.sum(alpha * feat, axis=0, keepdims=True)
    out_ref[...] = jnp.dot(hg, cw_ref[...],
                           preferred_element_type=jnp.float32) + cb_ref[...]


_full3 = pl.BlockSpec((NE, D, D), lambda i: (0, 0, 0))
_fullb = pl.BlockSpec((NE, 1, D), lambda i: (0, 0, 0))
_tout = pl.BlockSpec((NE, BN, H), lambda i: (0, i, 0))
_tshape = jax.ShapeDtypeStruct((NE, N_NODES, H), jnp.float32)

_t_call = pl.pallas_call(
    _t_body,
    grid=(NB,),
    in_specs=[pl.BlockSpec((BN, D), lambda i: (i, 0)), _full3, _fullb],
    out_specs=[_tout, _tout],
    out_shape=[_tshape, _tshape],
)

_gru_in_specs = [
    pl.BlockSpec((BN, D), lambda i: (i, 0)),
    pl.BlockSpec((2, BN, H), lambda i: (0, i, 0)),
    pl.BlockSpec((D, 3 * D), lambda i: (0, 0)),
    pl.BlockSpec((D, 3 * D), lambda i: (0, 0)),
    pl.BlockSpec((1, 3 * D), lambda i: (0, 0)),
    pl.BlockSpec((1, 3 * D), lambda i: (0, 0)),
]

_a_call = pl.pallas_call(
    _a_body,
    grid=(NB,),
    in_specs=_gru_in_specs + [_full3, _fullb],
    out_specs=[pl.BlockSpec((BN, D), lambda i: (i, 0)), _tout, _tout],
    out_shape=[jax.ShapeDtypeStruct((N_NODES, D), jnp.float32),
               _tshape, _tshape],
)

_g_call = pl.pallas_call(
    _g_body,
    grid=(NB,),
    in_specs=_gru_in_specs,
    out_specs=pl.BlockSpec((BN, D), lambda i: (i, 0)),
    out_shape=jax.ShapeDtypeStruct((N_NODES, D), jnp.float32),
)

_p_call = pl.pallas_call(
    _p_body,
    out_shape=jax.ShapeDtypeStruct((1, 1), jnp.float32),
)


def kernel(x, edge_index, etype, W, b, w_ih, w_hh, b_ih, b_hh,
           gate_w, gate_b, feat_w, feat_b, cls_w, cls_b):
    src = edge_index[0]
    dst = edge_index[1]
    key = etype * N_NODES + src
    pad = E_PAD - N_EDGES
    key_p = jnp.concatenate(
        [key, jnp.zeros((pad,), jnp.int32)]).reshape(N_SUBCORES, NCH, CH)
    dst_p = jnp.concatenate(
        [dst, jnp.full((pad,), N_NODES, jnp.int32)]).reshape(N_SUBCORES, NCH, CH)
    zeros_acc = jnp.zeros((N_ACC, H), jnp.float32)

    wt = jnp.transpose(W, (0, 2, 1))
    bt = b[:, None, :]
    wih_t = w_ih.T
    whh_t = w_hh.T
    bih2 = b_ih[None, :]
    bhh2 = b_hh[None, :]

    sc_scatter = _sc_scatter_call()
    t0, t1 = _t_call(x, wt, bt)
    h = x
    for step in range(N_STEPS):
        a = sc_scatter(key_p, dst_p,
                        t0.reshape(NE * N_NODES, H),
                        t1.reshape(NE * N_NODES, H),
                        zeros_acc)
        if step < N_STEPS - 1:
            h, t0, t1 = _a_call(h, a, wih_t, whh_t, bih2, bhh2, wt, bt)
        else:
            h = _g_call(h, a, wih_t, whh_t, bih2, bhh2)
    return _p_call(h, gate_w.T, gate_b[None, :], feat_w.T, feat_b[None, :],
                   cls_w.T, cls_b[None, :])


# speedup vs baseline: 1.1188x; 1.1188x over previous
import functools

import jax
import jax.numpy as jnp
from jax import lax
from jax.experimental import pallas as pl
from jax.experimental.pallas import tpu as pltpu
from jax.experimental.pallas import tpu_sc as plsc

N_NODES = 10000
N_EDGES = 160000
D = 256
H = 128
NE = 4
N_STEPS = 6

BN = 1000
NB = N_NODES // BN

N_SUBCORES = 16
CH = 128
NCH = -(-(N_EDGES // N_SUBCORES) // CH)
E_PAD = N_SUBCORES * NCH * CH
ZR = -(-(N_NODES + 1) // N_SUBCORES) // 8 * 8 + 8
N_ACC = N_SUBCORES * ZR


def _sc_body(key_hbm, dst_hbm, t0_hbm, t1_hbm, z_hbm, a_hbm,
             kbuf, dbuf, mbuf, acc, sem):
    c = lax.axis_index("c")
    s = lax.axis_index("s")
    pltpu.sync_copy(key_hbm.at[s], kbuf)
    pltpu.sync_copy(dst_hbm.at[s], dbuf)
    pltpu.sync_copy(z_hbm.at[pl.ds(s * ZR, ZR)], acc.at[pl.ds(s * ZR, ZR)])
    plsc.subcore_barrier()

    def run(t_hbm):
        def chunk(j, carry):
            pltpu.async_copy(t_hbm.at[kbuf.at[j]], mbuf, sem).wait()
            pltpu.sync_copy(mbuf, acc.at[dbuf.at[j]], add=True)
            return carry

        lax.fori_loop(0, NCH, chunk, 0)

    @pl.when(c == 0)
    def _():
        run(t0_hbm)

    @pl.when(c == 1)
    def _():
        run(t1_hbm)

    plsc.subcore_barrier()

    @pl.when(c == 0)
    def _():
        pltpu.sync_copy(acc.at[pl.ds(s * ZR, ZR)],
                        a_hbm.at[0].at[pl.ds(s * ZR, ZR)])

    @pl.when(c == 1)
    def _():
        pltpu.sync_copy(acc.at[pl.ds(s * ZR, ZR)],
                        a_hbm.at[1].at[pl.ds(s * ZR, ZR)])


@functools.lru_cache(maxsize=1)
def _sc_scatter_call():
    mesh = plsc.VectorSubcoreMesh(core_axis_name="c", subcore_axis_name="s")
    return pl.kernel(
        _sc_body,
        out_type=jax.ShapeDtypeStruct((2, N_ACC, H), jnp.float32),
        mesh=mesh,
        scratch_types=[
            pltpu.VMEM((NCH, CH), jnp.int32),
            pltpu.VMEM((NCH, CH), jnp.int32),
            pltpu.VMEM((CH, H), jnp.float32),
            pltpu.VMEM_SHARED((N_ACC, H), jnp.float32),
            pltpu.SemaphoreType.DMA,
        ],
    )


def _transform_write(hn, wt_ref, bt_ref, t0_ref, t1_ref):
    for k in range(NE):
        v = jnp.dot(hn, wt_ref[k], preferred_element_type=jnp.float32)
        v = v + bt_ref[k]
        t0_ref[k] = v[:, :H]
        t1_ref[k] = v[:, H:]


def _gru_compute(h, a0, a1, wih_ref, whh_ref, bih_ref, bhh_ref):
    a = jnp.concatenate([a0, a1], axis=1)
    gi = jnp.dot(a, wih_ref[...], preferred_element_type=jnp.float32) + bih_ref[...]
    gh = jnp.dot(h, whh_ref[...], preferred_element_type=jnp.float32) + bhh_ref[...]
    r = jax.nn.sigmoid(gi[:, :D] + gh[:, :D])
    z = jax.nn.sigmoid(gi[:, D:2 * D] + gh[:, D:2 * D])
    n = jnp.tanh(gi[:, 2 * D:] + r * gh[:, 2 * D:])
    return (1.0 - z) * n + z * h


def _t_body(x_ref, wt_ref, bt_ref, t0_ref, t1_ref):
    _transform_write(x_ref[...], wt_ref, bt_ref, t0_ref, t1_ref)


def _a_body(h_ref, a_ref, wih_ref, whh_ref, bih_ref, bhh_ref, wt_ref, bt_ref,
            h_out, t0_ref, t1_ref):
    hn = _gru_compute(h_ref[...], a_ref[0], a_ref[1],
                      wih_ref, whh_ref, bih_ref, bhh_ref)
    h_out[...] = hn
    _transform_write(hn, wt_ref, bt_ref, t0_ref, t1_ref)


def _g_body(h_ref, a_ref, wih_ref, whh_ref, bih_ref, bhh_ref, h_out):
    h_out[...] = _gru_compute(h_ref[...], a_ref[0], a_ref[1],
                              wih_ref, whh_ref, bih_ref, bhh_ref)


def _p_body(h_ref, gw_ref, gb_ref, fwt_ref, fb_ref, cw_ref, cb_ref, out_ref):
    h = h_ref[...]
    gate = jnp.dot(h, gw_ref[...], preferred_element_type=jnp.float32) + gb_ref[...]
    m = jnp.max(gate)
    e = jnp.exp(gate - m)
    alpha = e / jnp.sum(e)
    feat = jnp.dot(h, fwt_ref[...],
                   preferred_element_type=jnp.float32) + fb_ref[...]
    hg = jnp.sum(alpha * feat, axis=0, keepdims=True)
    out_ref[...] = jnp.dot(hg, cw_ref[...],
                           preferred_element_type=jnp.float32) + cb_ref[...]


_full3 = pl.BlockSpec((NE, D, D), lambda i: (0, 0, 0))
_fullb = pl.BlockSpec((NE, 1, D), lambda i: (0, 0, 0))
_tout = pl.BlockSpec((NE, BN, H), lambda i: (0, i, 0))
_tshape = jax.ShapeDtypeStruct((NE, N_NODES, H), jnp.float32)

_t_call = pl.pallas_call(
    _t_body,
    grid=(NB,),
    in_specs=[pl.BlockSpec((BN, D), lambda i: (i, 0)), _full3, _fullb],
    out_specs=[_tout, _tout],
    out_shape=[_tshape, _tshape],
)

_gru_in_specs = [
    pl.BlockSpec((BN, D), lambda i: (i, 0)),
    pl.BlockSpec((2, BN, H), lambda i: (0, i, 0)),
    pl.BlockSpec((D, 3 * D), lambda i: (0, 0)),
    pl.BlockSpec((D, 3 * D), lambda i: (0, 0)),
    pl.BlockSpec((1, 3 * D), lambda i: (0, 0)),
    pl.BlockSpec((1, 3 * D), lambda i: (0, 0)),
]

_a_call = pl.pallas_call(
    _a_body,
    grid=(NB,),
    in_specs=_gru_in_specs + [_full3, _fullb],
    out_specs=[pl.BlockSpec((BN, D), lambda i: (i, 0)), _tout, _tout],
    out_shape=[jax.ShapeDtypeStruct((N_NODES, D), jnp.float32),
               _tshape, _tshape],
)

_g_call = pl.pallas_call(
    _g_body,
    grid=(NB,),
    in_specs=_gru_in_specs,
    out_specs=pl.BlockSpec((BN, D), lambda i: (i, 0)),
    out_shape=jax.ShapeDtypeStruct((N_NODES, D), jnp.float32),
)

_p_call = pl.pallas_call(
    _p_body,
    out_shape=jax.ShapeDtypeStruct((1, 1), jnp.float32),
)


def kernel(x, edge_index, etype, W, b, w_ih, w_hh, b_ih, b_hh,
           gate_w, gate_b, feat_w, feat_b, cls_w, cls_b):
    src = edge_index[0]
    dst = edge_index[1]
    key = etype * N_NODES + src
    pad = E_PAD - N_EDGES
    key_p = jnp.concatenate(
        [key, jnp.zeros((pad,), jnp.int32)]).reshape(N_SUBCORES, NCH, CH)
    dst_p = jnp.concatenate(
        [dst, jnp.full((pad,), N_NODES, jnp.int32)]).reshape(N_SUBCORES, NCH, CH)
    zeros_acc = jnp.zeros((N_ACC, H), jnp.float32)

    wt = jnp.transpose(W, (0, 2, 1))
    bt = b[:, None, :]
    wih_t = w_ih.T
    whh_t = w_hh.T
    bih2 = b_ih[None, :]
    bhh2 = b_hh[None, :]

    sc_scatter = _sc_scatter_call()
    t0, t1 = _t_call(x, wt, bt)
    h = x
    for step in range(N_STEPS):
        a = sc_scatter(key_p, dst_p,
                        t0.reshape(NE * N_NODES, H),
                        t1.reshape(NE * N_NODES, H),
                        zeros_acc)
        if step < N_STEPS - 1:
            h, t0, t1 = _a_call(h, a, wih_t, whh_t, bih2, bhh2, wt, bt)
        else:
            h = _g_call(h, a, wih_t, whh_t, bih2, bhh2)
    return _p_call(h, gate_w.T, gate_b[None, :], feat_w.T, feat_b[None, :],
                   cls_w.T, cls_b[None, :])
